# baseline (speedup 1.0000x reference)
import functools

import jax
import jax.numpy as jnp
from jax import lax
from jax.experimental import pallas as pl
from jax.experimental.pallas import tpu as pltpu
from jax.experimental.pallas import tpu_sc as plsc

_N = 18432
_K = 1024
_D = 64
_BT = 1024
_RG = 8

_NH = _N // 2
_NBH = _NH // _BT

_NW = 32
_BPW = _NH // _NW
_IC = 96
_NCH = _BPW // _IC
_DP = 128


def _vq_tc_body(wt_ref, xt_ref, w2_ref, idx_ref, acc_ref):
    xt = xt_ref[...]
    mmt = lax.dot_general(
        wt_ref[...], xt,
        dimension_numbers=(((0,), (0,)), ((), ())),
        preferred_element_type=jnp.float32,
    )
    x2 = jnp.sum(xt * xt, axis=0, keepdims=True)
    w2 = w2_ref[...]
    d0 = (x2 + w2[0:_RG]) - 2.0 * mmt[0:_RG]
    run_min = d0
    run_tile = jnp.zeros(d0.shape, jnp.int32)
    for r in range(1, _K // _RG):
        dr = (x2 + w2[r * _RG:(r + 1) * _RG]) - 2.0 * mmt[r * _RG:(r + 1) * _RG]
        cond = dr < run_min
        run_min = jnp.where(cond, dr, run_min)
        run_tile = jnp.where(cond, r, run_tile)
    v = run_min
    i = run_tile * _RG + lax.broadcasted_iota(jnp.int32, d0.shape, 0)
    h = _RG
    while h > 1:
        h //= 2
        va, vb = v[0:h], v[h:2 * h]
        ia, ib = i[0:h], i[h:2 * h]
        cond = (vb < va) | ((vb == va) & (ib < ia))
        v = jnp.where(cond, vb, va)
        i = jnp.where(cond, ib, ia)
    idx_ref[...] = i.reshape(_BT)

    @pl.when(pl.program_id(0) == 0)
    def _init():
        acc_ref[...] = jnp.zeros_like(acc_ref)

    acc_ref[...] += jnp.sum(v, axis=1, keepdims=True)


def _make_tc_call(block_offset):
    return pl.pallas_call(
        _vq_tc_body,
        grid=(_NBH,),
        in_specs=[
            pl.BlockSpec((_D, _K), lambda i: (0, 0)),
            pl.BlockSpec((_D, _BT), lambda i: (0, i + block_offset)),
            pl.BlockSpec((_K, 1), lambda i: (0, 0)),
        ],
        out_specs=[
            pl.BlockSpec((_BT,), lambda i: (i,)),
            pl.BlockSpec((1, 1), lambda i: (0, 0)),
        ],
        out_shape=[
            jax.ShapeDtypeStruct((_NH,), jnp.int32),
            jax.ShapeDtypeStruct((1, 1), jnp.float32),
        ],
        compiler_params=pltpu.CompilerParams(dimension_semantics=("arbitrary",)),
    )


_tc_call_a = _make_tc_call(0)
_tc_call_b = _make_tc_call(_NBH)


@functools.lru_cache(maxsize=None)
def _make_sc_gather():
    mesh = plsc.VectorSubcoreMesh(core_axis_name="c", subcore_axis_name="s")

    @functools.partial(
        pl.kernel,
        mesh=mesh,
        out_type=jax.ShapeDtypeStruct((_NH, _DP), jnp.float32),
        scratch_types=[
            pltpu.VMEM((_BPW,), jnp.int32),
            pltpu.VMEM((_BPW, _DP), jnp.float32),
            pltpu.SemaphoreType.DMA,
        ],
    )
    def _sc_gather(w_hbm, idx_hbm, out_hbm, idx_v, rows_v, sem):
        wid = lax.axis_index("s") * 2 + lax.axis_index("c")
        base = wid * _BPW
        pltpu.sync_copy(idx_hbm.at[pl.ds(base, _BPW)], idx_v)
        copies = [
            pltpu.async_copy(
                w_hbm.at[idx_v.at[pl.ds(j * _IC, _IC)]],
                rows_v.at[pl.ds(j * _IC, _IC)],
                sem,
            )
            for j in range(_NCH)
        ]
        for c in copies:
            c.wait()
        pltpu.sync_copy(rows_v, out_hbm.at[pl.ds(base, _BPW)])

    return _sc_gather


def kernel(inputs, weight):
    wt = weight.T
    xt = inputs.T
    w2c = jnp.sum(weight ** 2, axis=1).reshape(_K, 1)
    w_pad = jnp.concatenate(
        [weight, jnp.zeros((_K, _DP - _D), jnp.float32)], axis=1)
    idx_a, acc_a = _tc_call_a(wt, xt, w2c)
    q_a = _make_sc_gather()(w_pad, idx_a)
    idx_b, acc_b = _tc_call_b(wt, xt, w2c)
    q_b = _make_sc_gather()(w_pad, idx_b)
    quantized = jnp.concatenate([q_a[:, :_D], q_b[:, :_D]], axis=0)
    idx = jnp.concatenate([idx_a, idx_b])
    lmean = (acc_a[0, 0] + acc_b[0, 0]) / (_N * _D)
    vq_loss = lmean + 0.25 * lmean
    return (vq_loss, quantized, idx)


# speedup vs baseline: 1.0824x; 1.0824x over previous
import functools

import jax
import jax.numpy as jnp
from jax import lax
from jax.experimental import pallas as pl
from jax.experimental.pallas import tpu as pltpu
from jax.experimental.pallas import tpu_sc as plsc

_N = 18432
_K = 1024
_D = 64
_BT = 1024
_NB = _N // _BT
_RG = 8

_NW = 32
_BPW = _N // _NW
_IC = 96
_NCH = _BPW // _IC
_DP = 128


def _vq_tc_body(wt_ref, xt_ref, w2_ref, idx_ref, acc_ref):
    xt = xt_ref[...]
    mmt = lax.dot_general(
        wt_ref[...], xt,
        dimension_numbers=(((0,), (0,)), ((), ())),
        preferred_element_type=jnp.float32,
    )
    x2 = jnp.sum(xt * xt, axis=0, keepdims=True)
    w2 = w2_ref[...]
    d0 = (x2 + w2[0:_RG]) - 2.0 * mmt[0:_RG]
    run_min = d0
    run_tile = jnp.zeros(d0.shape, jnp.int32)
    for r in range(1, _K // _RG):
        dr = (x2 + w2[r * _RG:(r + 1) * _RG]) - 2.0 * mmt[r * _RG:(r + 1) * _RG]
        cond = dr < run_min
        run_min = jnp.where(cond, dr, run_min)
        run_tile = jnp.where(cond, r, run_tile)
    v = run_min
    i = run_tile * _RG + lax.broadcasted_iota(jnp.int32, d0.shape, 0)
    h = _RG
    while h > 1:
        h //= 2
        va, vb = v[0:h], v[h:2 * h]
        ia, ib = i[0:h], i[h:2 * h]
        cond = (vb < va) | ((vb == va) & (ib < ia))
        v = jnp.where(cond, vb, va)
        i = jnp.where(cond, ib, ia)
    idx_ref[...] = i.reshape(_BT)

    @pl.when(pl.program_id(0) == 0)
    def _init():
        acc_ref[...] = jnp.zeros_like(acc_ref)

    acc_ref[...] += jnp.sum(v, axis=1, keepdims=True)


_tc_call = pl.pallas_call(
    _vq_tc_body,
    grid=(_NB,),
    in_specs=[
        pl.BlockSpec((_D, _K), lambda i: (0, 0)),
        pl.BlockSpec((_D, _BT), lambda i: (0, i)),
        pl.BlockSpec((_K, 1), lambda i: (0, 0)),
    ],
    out_specs=[
        pl.BlockSpec((_BT,), lambda i: (i,)),
        pl.BlockSpec((1, 1), lambda i: (0, 0)),
    ],
    out_shape=[
        jax.ShapeDtypeStruct((_N,), jnp.int32),
        jax.ShapeDtypeStruct((1, 1), jnp.float32),
    ],
    compiler_params=pltpu.CompilerParams(dimension_semantics=("arbitrary",)),
)


@functools.lru_cache(maxsize=None)
def _make_sc_gather():
    mesh = plsc.VectorSubcoreMesh(core_axis_name="c", subcore_axis_name="s")

    @functools.partial(
        pl.kernel,
        mesh=mesh,
        out_type=jax.ShapeDtypeStruct((_N, _DP), jnp.float32),
        scratch_types=[
            pltpu.VMEM((_BPW,), jnp.int32),
            pltpu.VMEM((_BPW, _DP), jnp.float32),
            pltpu.SemaphoreType.DMA,
            pltpu.SemaphoreType.DMA,
        ],
    )
    def _sc_gather(w_hbm, idx_hbm, out_hbm, idx_v, rows_v, osem, sem):
        wid = lax.axis_index("s") * 2 + lax.axis_index("c")
        base = wid * _BPW
        pltpu.sync_copy(idx_hbm.at[pl.ds(base, _BPW)], idx_v)
        copies = [
            pltpu.async_copy(
                w_hbm.at[idx_v.at[pl.ds(j * _IC, _IC)]],
                rows_v.at[pl.ds(j * _IC, _IC)],
                sem,
            )
            for j in range(_NCH)
        ]
        outs = []
        for j, c in enumerate(copies):
            c.wait()
            outs.append(
                pltpu.async_copy(
                    rows_v.at[pl.ds(j * _IC, _IC)],
                    out_hbm.at[pl.ds(base + j * _IC, _IC)],
                    osem,
                )
            )
        for o in outs:
            o.wait()

    return _sc_gather


def kernel(inputs, weight):
    wt = weight.T
    w2c = jnp.sum(wt * wt, axis=0).reshape(_K, 1)
    w_pad = jnp.concatenate(
        [weight, jnp.zeros((_K, _DP - _D), jnp.float32)], axis=1)
    idx, acc = _tc_call(wt, inputs.T, w2c)
    quantized = _make_sc_gather()(w_pad, idx)[:, :_D]
    lmean = acc[0, 0] / (_N * _D)
    vq_loss = lmean + 0.25 * lmean
    return (vq_loss, quantized, idx)


# speedup vs baseline: 1.1282x; 1.0423x over previous
import functools

import jax
import jax.numpy as jnp
from jax import lax
from jax.experimental import pallas as pl
from jax.experimental.pallas import tpu as pltpu
from jax.experimental.pallas import tpu_sc as plsc

_N = 18432
_K = 1024
_D = 64
_BT = 2048
_NB = _N // _BT
_RG = 8

_NW = 32
_BPW = _N // _NW
_IC = 96
_NCH = _BPW // _IC
_DP = 128


def _vq_tc_body(wt_ref, xt_ref, w2_ref, idx_ref, acc_ref):
    xt = xt_ref[...]
    mmt = lax.dot_general(
        wt_ref[...], xt,
        dimension_numbers=(((0,), (0,)), ((), ())),
        preferred_element_type=jnp.float32,
    )
    x2 = jnp.sum(xt * xt, axis=0, keepdims=True)
    w2 = w2_ref[...]
    d0 = (x2 + w2[0:_RG]) - 2.0 * mmt[0:_RG]
    run_min = d0
    run_tile = jnp.zeros(d0.shape, jnp.int32)
    for r in range(1, _K // _RG):
        dr = (x2 + w2[r * _RG:(r + 1) * _RG]) - 2.0 * mmt[r * _RG:(r + 1) * _RG]
        cond = dr < run_min
        run_min = jnp.where(cond, dr, run_min)
        run_tile = jnp.where(cond, r, run_tile)
    v = run_min
    i = run_tile * _RG + lax.broadcasted_iota(jnp.int32, d0.shape, 0)
    h = _RG
    while h > 1:
        h //= 2
        va, vb = v[0:h], v[h:2 * h]
        ia, ib = i[0:h], i[h:2 * h]
        cond = (vb < va) | ((vb == va) & (ib < ia))
        v = jnp.where(cond, vb, va)
        i = jnp.where(cond, ib, ia)
    idx_ref[...] = i.reshape(_BT)

    @pl.when(pl.program_id(0) == 0)
    def _init():
        acc_ref[...] = jnp.zeros_like(acc_ref)

    acc_ref[...] += jnp.sum(v, axis=1, keepdims=True)


_tc_call = pl.pallas_call(
    _vq_tc_body,
    grid=(_NB,),
    in_specs=[
        pl.BlockSpec((_D, _K), lambda i: (0, 0)),
        pl.BlockSpec((_D, _BT), lambda i: (0, i)),
        pl.BlockSpec((_K, 1), lambda i: (0, 0)),
    ],
    out_specs=[
        pl.BlockSpec((_BT,), lambda i: (i,)),
        pl.BlockSpec((1, 1), lambda i: (0, 0)),
    ],
    out_shape=[
        jax.ShapeDtypeStruct((_N,), jnp.int32),
        jax.ShapeDtypeStruct((1, 1), jnp.float32),
    ],
    compiler_params=pltpu.CompilerParams(dimension_semantics=("arbitrary",)),
)


@functools.lru_cache(maxsize=None)
def _make_sc_gather():
    mesh = plsc.VectorSubcoreMesh(core_axis_name="c", subcore_axis_name="s")

    @functools.partial(
        pl.kernel,
        mesh=mesh,
        out_type=jax.ShapeDtypeStruct((_N, _DP), jnp.float32),
        scratch_types=[
            pltpu.VMEM((_BPW,), jnp.int32),
            pltpu.VMEM((_BPW, _DP), jnp.float32),
            pltpu.SemaphoreType.DMA,
        ],
    )
    def _sc_gather(w_hbm, idx_hbm, out_hbm, idx_v, rows_v, sem):
        wid = lax.axis_index("s") * 2 + lax.axis_index("c")
        base = wid * _BPW
        pltpu.sync_copy(idx_hbm.at[pl.ds(base, _BPW)], idx_v)
        copies = [
            pltpu.async_copy(
                w_hbm.at[idx_v.at[pl.ds(j * _IC, _IC)]],
                rows_v.at[pl.ds(j * _IC, _IC)],
                sem,
            )
            for j in range(_NCH)
        ]
        for c in copies:
            c.wait()
        pltpu.sync_copy(rows_v, out_hbm.at[pl.ds(base, _BPW)])

    return _sc_gather


def kernel(inputs, weight):
    w2c = jnp.sum(weight ** 2, axis=1).reshape(_K, 1)
    w_pad = jnp.concatenate(
        [weight, jnp.zeros((_K, _DP - _D), jnp.float32)], axis=1)
    idx, acc = _tc_call(weight.T, inputs.T, w2c)
    quantized = _make_sc_gather()(w_pad, idx)[:, :_D]
    lmean = acc[0, 0] / (_N * _D)
    vq_loss = lmean + 0.25 * lmean
    return (vq_loss, quantized, idx)


# speedup vs baseline: 1.1401x; 1.0105x over previous
import functools

import jax
import jax.numpy as jnp
from jax import lax
from jax.experimental import pallas as pl
from jax.experimental.pallas import tpu as pltpu
from jax.experimental.pallas import tpu_sc as plsc

_N = 18432
_K = 1024
_D = 64
_BT = 3072
_NB = _N // _BT
_RG = 8

_NW = 32
_BPW = _N // _NW
_IC = 96
_NCH = _BPW // _IC
_DP = 128


def _vq_tc_body(wt_ref, xt_ref, w2_ref, idx_ref, acc_ref):
    xt = xt_ref[...]
    mmt = lax.dot_general(
        wt_ref[...], xt,
        dimension_numbers=(((0,), (0,)), ((), ())),
        preferred_element_type=jnp.float32,
    )
    x2 = jnp.sum(xt * xt, axis=0, keepdims=True)
    w2 = w2_ref[...]
    d0 = (x2 + w2[0:_RG]) - 2.0 * mmt[0:_RG]
    run_min = d0
    run_tile = jnp.zeros(d0.shape, jnp.int32)
    for r in range(1, _K // _RG):
        dr = (x2 + w2[r * _RG:(r + 1) * _RG]) - 2.0 * mmt[r * _RG:(r + 1) * _RG]
        cond = dr < run_min
        run_min = jnp.where(cond, dr, run_min)
        run_tile = jnp.where(cond, r, run_tile)
    v = run_min
    i = run_tile * _RG + lax.broadcasted_iota(jnp.int32, d0.shape, 0)
    h = _RG
    while h > 1:
        h //= 2
        va, vb = v[0:h], v[h:2 * h]
        ia, ib = i[0:h], i[h:2 * h]
        cond = (vb < va) | ((vb == va) & (ib < ia))
        v = jnp.where(cond, vb, va)
        i = jnp.where(cond, ib, ia)
    idx_ref[...] = i.reshape(_BT)

    @pl.when(pl.program_id(0) == 0)
    def _init():
        acc_ref[...] = jnp.zeros_like(acc_ref)

    acc_ref[...] += jnp.sum(v, axis=1, keepdims=True)


_tc_call = pl.pallas_call(
    _vq_tc_body,
    grid=(_NB,),
    in_specs=[
        pl.BlockSpec((_D, _K), lambda i: (0, 0)),
        pl.BlockSpec((_D, _BT), lambda i: (0, i)),
        pl.BlockSpec((_K, 1), lambda i: (0, 0)),
    ],
    out_specs=[
        pl.BlockSpec((_BT,), lambda i: (i,)),
        pl.BlockSpec((1, 1), lambda i: (0, 0)),
    ],
    out_shape=[
        jax.ShapeDtypeStruct((_N,), jnp.int32),
        jax.ShapeDtypeStruct((1, 1), jnp.float32),
    ],
    compiler_params=pltpu.CompilerParams(dimension_semantics=("arbitrary",)),
)


@functools.lru_cache(maxsize=None)
def _make_sc_gather():
    mesh = plsc.VectorSubcoreMesh(core_axis_name="c", subcore_axis_name="s")

    @functools.partial(
        pl.kernel,
        mesh=mesh,
        out_type=jax.ShapeDtypeStruct((_N, _DP), jnp.float32),
        scratch_types=[
            pltpu.VMEM((_BPW,), jnp.int32),
            pltpu.VMEM((_BPW, _DP), jnp.float32),
            pltpu.SemaphoreType.DMA,
        ],
    )
    def _sc_gather(w_hbm, idx_hbm, out_hbm, idx_v, rows_v, sem):
        wid = lax.axis_index("s") * 2 + lax.axis_index("c")
        base = wid * _BPW
        pltpu.sync_copy(idx_hbm.at[pl.ds(base, _BPW)], idx_v)
        copies = [
            pltpu.async_copy(
                w_hbm.at[idx_v.at[pl.ds(j * _IC, _IC)]],
                rows_v.at[pl.ds(j * _IC, _IC)],
                sem,
            )
            for j in range(_NCH)
        ]
        for c in copies:
            c.wait()
        pltpu.sync_copy(rows_v, out_hbm.at[pl.ds(base, _BPW)])

    return _sc_gather


def kernel(inputs, weight):
    w2c = jnp.sum(weight ** 2, axis=1).reshape(_K, 1)
    w_pad = jnp.concatenate(
        [weight, jnp.zeros((_K, _DP - _D), jnp.float32)], axis=1)
    idx, acc = _tc_call(weight.T, inputs.T, w2c)
    quantized = _make_sc_gather()(w_pad, idx)[:, :_D]
    lmean = acc[0, 0] / (_N * _D)
    vq_loss = lmean + 0.25 * lmean
    return (vq_loss, quantized, idx)
